# baseline (speedup 1.0000x reference)
import functools

import jax
import jax.numpy as jnp
from jax import lax
from jax.experimental import pallas as pl
from jax.experimental.pallas import tpu as pltpu
from jax.experimental.pallas import tpu_sc as plsc

_NC = 2
_NS = 16
_NW = _NC * _NS


def kernel(cqt, pos_cqt, channel_idx):
    B, C, F, T = cqt.shape
    idx32 = channel_idx.astype(jnp.int32)
    b_per_w = B // _NW

    mesh = plsc.VectorSubcoreMesh(core_axis_name="c", subcore_axis_name="s")

    @functools.partial(
        pl.kernel,
        out_type=jax.ShapeDtypeStruct(cqt.shape, cqt.dtype),
        mesh=mesh,
        scratch_types=[pltpu.MemorySpace.VMEM((B + 16,), jnp.int32)],
    )
    def k(cqt_hbm, pos_hbm, idx_hbm, out_hbm, idx_v):
        wid = lax.axis_index("s") * _NC + lax.axis_index("c")
        pltpu.sync_copy(idx_hbm, idx_v.at[pl.ds(0, B)])
        vec = idx_v[pl.ds(wid * b_per_w, 16)]
        for j in range(b_per_w):
            b = wid * b_per_w + j
            sel = vec[j]
            for c in range(C):
                @pl.when(sel == c)
                def _():
                    pltpu.sync_copy(pos_hbm.at[b, c], out_hbm.at[b, c])

                @pl.when(sel != c)
                def _():
                    pltpu.sync_copy(cqt_hbm.at[b, c], out_hbm.at[b, c])

    return k(cqt, pos_cqt, idx32)


# speedup vs baseline: 8.9123x; 8.9123x over previous
import functools

import jax
import jax.numpy as jnp
from jax import lax
from jax.experimental import pallas as pl
from jax.experimental.pallas import tpu as pltpu
from jax.experimental.pallas import tpu_sc as plsc

_NC = 2
_NS = 16
_NW = _NC * _NS


def kernel(cqt, pos_cqt, channel_idx):
    B, C, F, T = cqt.shape
    idx32 = channel_idx.astype(jnp.int32)
    b_per_w = B // _NW
    n_slabs = b_per_w * C

    mesh = plsc.VectorSubcoreMesh(core_axis_name="c", subcore_axis_name="s")

    @functools.partial(
        pl.kernel,
        out_type=jax.ShapeDtypeStruct(cqt.shape, cqt.dtype),
        mesh=mesh,
        scratch_types=[
            pltpu.MemorySpace.VMEM((B + 16,), jnp.int32),
            pltpu.MemorySpace.VMEM((F, T), cqt.dtype),
            pltpu.MemorySpace.VMEM((F, T), cqt.dtype),
            pltpu.SemaphoreType.DMA,
            pltpu.SemaphoreType.DMA,
            pltpu.SemaphoreType.DMA,
            pltpu.SemaphoreType.DMA,
        ],
    )
    def k(cqt_hbm, pos_hbm, idx_hbm, out_hbm, idx_v, buf0, buf1, g0, g1, s0, s1):
        wid = lax.axis_index("s") * _NC + lax.axis_index("c")
        pltpu.sync_copy(idx_hbm, idx_v.at[pl.ds(0, B)])
        vec = idx_v[pl.ds(wid * b_per_w, 16)]
        bufs = (buf0, buf1)
        gsems = (g0, g1)
        ssems = (s0, s1)

        def bc(i):
            return wid * b_per_w + i // C, i % C

        def gather_start(i):
            b, c = bc(i)
            sel = vec[i // C]
            buf, sem = bufs[i % 2], gsems[i % 2]

            @pl.when(sel == c)
            def _():
                pltpu.make_async_copy(pos_hbm.at[b, c], buf, sem).start()

            @pl.when(sel != c)
            def _():
                pltpu.make_async_copy(cqt_hbm.at[b, c], buf, sem).start()

        def gather_wait(i):
            b, c = bc(i)
            pltpu.make_async_copy(cqt_hbm.at[b, c], bufs[i % 2], gsems[i % 2]).wait()

        def scatter_start(i):
            b, c = bc(i)
            pltpu.make_async_copy(bufs[i % 2], out_hbm.at[b, c], ssems[i % 2]).start()

        def scatter_wait(i):
            b, c = bc(i)
            pltpu.make_async_copy(bufs[i % 2], out_hbm.at[b, c], ssems[i % 2]).wait()

        gather_start(0)
        for i in range(n_slabs):
            gather_wait(i)
            scatter_start(i)
            if i + 1 < n_slabs:
                if i >= 1:
                    scatter_wait(i - 1)
                gather_start(i + 1)
        scatter_wait(n_slabs - 2)
        scatter_wait(n_slabs - 1)

    return k(cqt, pos_cqt, idx32)
